# baseline (speedup 1.0000x reference)
import functools

import jax
import jax.numpy as jnp
from jax import lax
from jax.experimental import pallas as pl
from jax.experimental.pallas import tpu as pltpu
from jax.experimental.pallas import tpu_sc as plsc

N_CLASSES = 1000
N_CTX = 8
NAME_LEN = 77
OUT_LEN = N_CTX + NAME_LEN
CTX_DIM = 512

CTX_W = N_CTX * CTX_DIM
NAME_W = NAME_LEN * CTX_DIM
OUT_W = OUT_LEN * CTX_DIM

NW = 32
RD = 3


def kernel(ctx, name_embeds):
    mesh = plsc.VectorSubcoreMesh(core_axis_name="c", subcore_axis_name="s")

    @functools.partial(
        pl.kernel,
        mesh=mesh,
        out_type=jax.ShapeDtypeStruct((N_CLASSES, OUT_W), jnp.float32),
        scratch_types=[
            pltpu.VMEM((RD * CTX_W,), jnp.float32),
            pltpu.VMEM((RD * NAME_W,), jnp.float32),
            pltpu.SemaphoreType.DMA((RD, 2)),
            pltpu.SemaphoreType.DMA((RD, 2)),
        ],
    )
    def _sc_concat(ctx_hbm, name_hbm, out_hbm, cbuf, nbuf, gsems, ssems):
        wid = lax.axis_index("s") * 2 + lax.axis_index("c")
        n = jnp.where(wid < 8, 32, 31)
        base = 31 * wid + jnp.minimum(wid, 8)

        def gathers(c, slot):
            g1 = pltpu.make_async_copy(
                ctx_hbm.at[c], cbuf.at[pl.ds(slot * CTX_W, CTX_W)], gsems.at[slot, 0]
            )
            g2 = pltpu.make_async_copy(
                name_hbm.at[c], nbuf.at[pl.ds(slot * NAME_W, NAME_W)], gsems.at[slot, 1]
            )
            return g1, g2

        def scatters(c, slot):
            s1 = pltpu.make_async_copy(
                cbuf.at[pl.ds(slot * CTX_W, CTX_W)],
                out_hbm.at[c, pl.ds(0, CTX_W)],
                ssems.at[slot, 0],
            )
            s2 = pltpu.make_async_copy(
                nbuf.at[pl.ds(slot * NAME_W, NAME_W)],
                out_hbm.at[c, pl.ds(CTX_W, NAME_W)],
                ssems.at[slot, 1],
            )
            return s1, s2

        def body(k, _):
            slot = k % RD

            @pl.when(k >= RD)
            def _():
                s1, s2 = scatters(base + k - RD, slot)
                s1.wait()
                s2.wait()

            @pl.when(k < n)
            def _():
                g1, g2 = gathers(base + k, slot)
                g1.start()
                g2.start()

            @pl.when((k >= 1) & (k <= n))
            def _():
                prev = (k - 1) % RD
                g1, g2 = gathers(base + k - 1, prev)
                g1.wait()
                g2.wait()
                s1, s2 = scatters(base + k - 1, prev)
                s1.start()
                s2.start()

            return 0

        lax.fori_loop(0, n + 2, body, 0)
        s1, s2 = scatters(base + n - 1, (n - 1) % RD)
        s1.wait()
        s2.wait()

    out_flat = _sc_concat(
        ctx.reshape(N_CLASSES, CTX_W), name_embeds.reshape(N_CLASSES, NAME_W)
    )
    return out_flat.reshape(N_CLASSES, OUT_LEN, CTX_DIM)


# speedup vs baseline: 1.7389x; 1.7389x over previous
import functools

import jax
import jax.numpy as jnp
from jax import lax
from jax.experimental import pallas as pl
from jax.experimental.pallas import tpu as pltpu
from jax.experimental.pallas import tpu_sc as plsc

N_CLASSES = 1000
N_CTX = 8
NAME_LEN = 77
OUT_LEN = N_CTX + NAME_LEN
CTX_DIM = 512

NW = 32
UB = 2


def kernel(ctx, name_embeds):
    mesh = plsc.VectorSubcoreMesh(core_axis_name="c", subcore_axis_name="s")

    @functools.partial(
        pl.kernel,
        mesh=mesh,
        out_type=jax.ShapeDtypeStruct((N_CLASSES, OUT_LEN, CTX_DIM), jnp.float32),
        scratch_types=[
            pltpu.VMEM((UB, OUT_LEN, CTX_DIM), jnp.float32),
            pltpu.SemaphoreType.DMA((2,)),
            pltpu.SemaphoreType.DMA((1,)),
        ],
    )
    def _sc_concat(ctx_hbm, name_hbm, out_hbm, buf, gsems, ssems):
        wid = lax.axis_index("s") * 2 + lax.axis_index("c")
        n = jnp.where(wid < 20, 16, 15)
        base = 15 * wid + jnp.minimum(wid, 20)

        def gathers(c0):
            g1 = pltpu.make_async_copy(
                ctx_hbm.at[pl.ds(c0, UB)], buf.at[:, pl.ds(0, N_CTX)], gsems.at[0]
            )
            g2 = pltpu.make_async_copy(
                name_hbm.at[pl.ds(c0, UB)],
                buf.at[:, pl.ds(N_CTX, NAME_LEN)],
                gsems.at[1],
            )
            return g1, g2

        def scatter(c0):
            return pltpu.make_async_copy(buf, out_hbm.at[pl.ds(c0, UB)], ssems.at[0])

        def body(k, _):
            c0 = (base + k) * UB
            g1, g2 = gathers(c0)
            g1.start()
            g2.start()
            g1.wait()
            g2.wait()
            s = scatter(c0)
            s.start()
            s.wait()
            return 0

        lax.fori_loop(0, n, body, 0)

    return _sc_concat(ctx, name_embeds)


# speedup vs baseline: 1.7843x; 1.0261x over previous
import functools

import jax
import jax.numpy as jnp
from jax import lax
from jax.experimental import pallas as pl
from jax.experimental.pallas import tpu as pltpu
from jax.experimental.pallas import tpu_sc as plsc

N_CLASSES = 1000
N_CTX = 8
NAME_LEN = 77
OUT_LEN = N_CTX + NAME_LEN
CTX_DIM = 512

B = 10
RD = 4
NCHUNKS = N_CLASSES // B
PER_CORE = NCHUNKS // 2


def kernel(ctx, name_embeds):
    mesh = plsc.ScalarSubcoreMesh(axis_name="c", num_cores=2)

    @functools.partial(
        pl.kernel,
        mesh=mesh,
        out_type=jax.ShapeDtypeStruct((N_CLASSES, OUT_LEN, CTX_DIM), jnp.float32),
        scratch_types=[
            pltpu.VMEM_SHARED((RD, B, OUT_LEN, CTX_DIM), jnp.float32),
            pltpu.SemaphoreType.DMA((RD, 2)),
            pltpu.SemaphoreType.DMA((RD,)),
        ],
    )
    def _sc_concat(ctx_hbm, name_hbm, out_hbm, buf, gsems, ssems):
        cid = lax.axis_index("c")
        base = cid * PER_CORE

        def gathers(chunk, slot):
            c0 = chunk * B
            g1 = pltpu.make_async_copy(
                ctx_hbm.at[pl.ds(c0, B)],
                buf.at[slot, :, pl.ds(0, N_CTX)],
                gsems.at[slot, 0],
            )
            g2 = pltpu.make_async_copy(
                name_hbm.at[pl.ds(c0, B)],
                buf.at[slot, :, pl.ds(N_CTX, NAME_LEN)],
                gsems.at[slot, 1],
            )
            return g1, g2

        def scatter(chunk, slot):
            return pltpu.make_async_copy(
                buf.at[slot], out_hbm.at[pl.ds(chunk * B, B)], ssems.at[slot]
            )

        def body(k, _):
            slot = k % RD

            @pl.when(k >= RD)
            def _():
                scatter(base + k - RD, slot).wait()

            @pl.when(k < PER_CORE)
            def _():
                g1, g2 = gathers(base + k, slot)
                g1.start()
                g2.start()

            @pl.when((k >= 1) & (k <= PER_CORE))
            def _():
                prev = (k - 1) % RD
                g1, g2 = gathers(base + k - 1, prev)
                g1.wait()
                g2.wait()
                scatter(base + k - 1, prev).start()

            return 0

        lax.fori_loop(0, PER_CORE + 2, body, 0)
        scatter(base + PER_CORE - 2, (PER_CORE - 2) % RD).wait()
        scatter(base + PER_CORE - 1, (PER_CORE - 1) % RD).wait()

    return _sc_concat(ctx, name_embeds)
